# baseline (speedup 1.0000x reference)
import functools
import math

import jax
import jax.numpy as jnp
from jax import lax
from jax.experimental import pallas as pl
from jax.experimental.pallas import tpu as pltpu
from jax.experimental.pallas import tpu_sc as plsc

NUM_TABLES = 26
VOCAB = 100000
EMB_DIM = 32
OFFSET = 13
B, S = 4096, 50
SCALE = math.sqrt(EMB_DIM)

NC, NS = 2, 16
NW = NC * NS
TB = 512
NCHUNK = B // TB
NTASK = S * NUM_TABLES * NCHUNK
TASKS_PER_W = NTASK // NW
D_EMB = NUM_TABLES * EMB_DIM


def _build_sc_kernel():
    mesh = plsc.VectorSubcoreMesh(core_axis_name="c", subcore_axis_name="s")

    @functools.partial(
        pl.kernel,
        mesh=mesh,
        out_type=jax.ShapeDtypeStruct((S, D_EMB, B), jnp.float32),
        compiler_params=pltpu.CompilerParams(
            use_tc_tiling_on_sc=False, needs_layout_passes=False
        ),
        scratch_types=[
            pltpu.VMEM((2 * TB,), jnp.int32),
            pltpu.VMEM((2 * TB, EMB_DIM), jnp.float32),
            pltpu.VMEM((EMB_DIM, 2 * TB), jnp.float32),
            pltpu.SemaphoreType.DMA,
            pltpu.SemaphoreType.DMA,
        ],
    )
    def k(tables_hbm, idx_hbm, out_hbm, idx_v, stage, tb_v, sem_g, sem_o):
        wid = lax.axis_index("s") * NC + lax.axis_index("c")
        tid0 = wid * TASKS_PER_W
        lane = lax.iota(jnp.int32, 16)
        e_idx = [lane + 16 * h for h in (0, 1)]

        def task_coords(tid):
            st = tid // NCHUNK
            c = tid % NCHUNK
            t = st % NUM_TABLES
            s = st // NUM_TABLES
            return s, t, c * TB

        def issue_gathers(tid, sel):
            s, t, b0 = task_coords(tid)
            pltpu.sync_copy(
                idx_hbm.at[pl.ds((s * NUM_TABLES + t) * B + b0, TB)],
                idx_v.at[pl.ds(sel * TB, TB)],
            )
            for j in range(TB // 128):
                pltpu.async_copy(
                    tables_hbm.at[t].at[
                        idx_v.at[pl.ds(sel * TB + j * 128, 128)]
                    ],
                    stage.at[pl.ds(sel * TB + j * 128, 128)],
                    sem_g,
                )

        def drain_gather(sel):
            pltpu.make_async_copy(
                tables_hbm.at[0].at[pl.ds(0, TB)],
                stage.at[pl.ds(sel * TB, TB)],
                sem_g,
            ).wait()

        def drain_out(sel):
            pltpu.make_async_copy(
                out_hbm.at[0, pl.ds(0, EMB_DIM), pl.ds(0, TB)],
                tb_v.at[:, pl.ds(sel * TB, TB)],
                sem_o,
            ).wait()

        issue_gathers(tid0, 0)

        def body(kk, carry):
            sel = lax.rem(kk, 2)
            @pl.when(kk < TASKS_PER_W - 1)
            def _():
                issue_gathers(tid0 + kk + 1, 1 - sel)

            drain_gather(sel)

            @pl.when(kk >= 2)
            def _():
                drain_out(sel)

            def row_body(r, rcarry):
                row = sel * TB + r
                b_col = jnp.full((16,), row, jnp.int32)
                for h in (0, 1):
                    v = stage[row, pl.ds(16 * h, 16)] * SCALE
                    plsc.store_scatter(tb_v, [e_idx[h], b_col], v)
                return rcarry

            lax.fori_loop(0, TB, row_body, 0)

            s, t, b0 = task_coords(tid0 + kk)
            pltpu.async_copy(
                tb_v.at[:, pl.ds(sel * TB, TB)],
                out_hbm.at[s, pl.ds(t * EMB_DIM, EMB_DIM), pl.ds(b0, TB)],
                sem_o,
            )
            return carry

        lax.fori_loop(0, TASKS_PER_W, body, 0)

        drain_out((TASKS_PER_W - 2) % 2)
        drain_out((TASKS_PER_W - 1) % 2)

    return k


def kernel(x, tables):
    cont_t = jnp.transpose(x[:, :, :OFFSET], (1, 2, 0))
    idx_t = jnp.transpose(x[:, :, OFFSET:].astype(jnp.int32), (1, 2, 0))
    idx = idx_t.reshape(-1)
    emb_t = _build_sc_kernel()(tables, idx)
    full = jnp.concatenate([cont_t, emb_t], axis=1)
    return jnp.transpose(full, (2, 0, 1))


# speedup vs baseline: 1.1663x; 1.1663x over previous
import functools
import math

import jax
import jax.numpy as jnp
from jax import lax
from jax.experimental import pallas as pl
from jax.experimental.pallas import tpu as pltpu
from jax.experimental.pallas import tpu_sc as plsc

NUM_TABLES = 26
VOCAB = 100000
EMB_DIM = 32
OFFSET = 13
B, S = 4096, 50
SCALE = math.sqrt(EMB_DIM)

NC, NS = 2, 16
NW = NC * NS
TB = 512
LB = TB // 128
NCHUNK = B // TB
NTASK = S * NUM_TABLES * NCHUNK
TASKS_PER_W = NTASK // NW
D_EMB = NUM_TABLES * EMB_DIM
UNROLL = 8


def _build_sc_kernel():
    mesh = plsc.VectorSubcoreMesh(core_axis_name="c", subcore_axis_name="s")

    @functools.partial(
        pl.kernel,
        mesh=mesh,
        out_type=jax.ShapeDtypeStruct(
            (S * D_EMB // 8, B // 128, 8, 128), jnp.float32
        ),
        compiler_params=pltpu.CompilerParams(
            use_tc_tiling_on_sc=False, needs_layout_passes=False
        ),
        scratch_types=[
            pltpu.VMEM((2 * TB,), jnp.int32),
            pltpu.VMEM((2 * TB, EMB_DIM), jnp.float32),
            pltpu.VMEM((4, 2 * LB, 8, 128), jnp.float32),
            pltpu.SemaphoreType.DMA,
            pltpu.SemaphoreType.DMA,
        ],
    )
    def k(tables_hbm, idx_hbm, out_hbm, idx_v, stage, tb_v, sem_g, sem_o):
        wid = lax.axis_index("s") * NC + lax.axis_index("c")
        tid0 = wid * TASKS_PER_W
        lane = lax.iota(jnp.int32, 16)
        i_idx = [(lane + 16 * h) // 8 for h in (0, 1)]
        drow_idx = lax.rem(lane, 8)

        def task_coords(tid):
            st = tid // NCHUNK
            c = tid % NCHUNK
            t = st % NUM_TABLES
            s = st // NUM_TABLES
            return s, t, c

        def issue_gathers(tid, sel):
            s, t, c = task_coords(tid)
            pltpu.sync_copy(
                idx_hbm.at[pl.ds((s * NUM_TABLES + t) * B + c * TB, TB)],
                idx_v.at[pl.ds(sel * TB, TB)],
            )
            for j in range(TB // 128):
                pltpu.async_copy(
                    tables_hbm.at[t].at[
                        idx_v.at[pl.ds(sel * TB + j * 128, 128)]
                    ],
                    stage.at[pl.ds(sel * TB + j * 128, 128)],
                    sem_g,
                )

        def drain_gather(sel):
            pltpu.make_async_copy(
                tables_hbm.at[0].at[pl.ds(0, TB)],
                stage.at[pl.ds(sel * TB, TB)],
                sem_g,
            ).wait()

        def drain_out(sel):
            pltpu.make_async_copy(
                out_hbm.at[pl.ds(0, 4), pl.ds(0, LB), :, :],
                tb_v.at[:, pl.ds(sel * LB, LB), :, :],
                sem_o,
            ).wait()

        issue_gathers(tid0, 0)

        def body(kk, carry):
            sel = lax.rem(kk, 2)
            @pl.when(kk < TASKS_PER_W - 1)
            def _():
                issue_gathers(tid0 + kk + 1, 1 - sel)

            drain_gather(sel)

            @pl.when(kk >= 2)
            def _():
                drain_out(sel)

            def row_body(rr, rcarry):
                for u in range(UNROLL):
                    r = rr * UNROLL + u
                    row = sel * TB + r
                    d1 = jnp.full((16,), sel * LB + r // 128, jnp.int32)
                    d2 = jnp.full((16,), lax.rem(r, 128), jnp.int32)
                    for h in (0, 1):
                        v = stage[row, pl.ds(16 * h, 16)] * SCALE
                        plsc.store_scatter(
                            tb_v, [i_idx[h], d1, drow_idx, d2], v
                        )
                return rcarry

            lax.fori_loop(0, TB // UNROLL, row_body, 0)

            s, t, c = task_coords(tid0 + kk)
            pltpu.async_copy(
                tb_v.at[:, pl.ds(sel * LB, LB), :, :],
                out_hbm.at[
                    pl.ds((s * D_EMB + t * EMB_DIM) // 8, 4),
                    pl.ds(c * LB, LB),
                    :,
                    :,
                ],
                sem_o,
            )
            return carry

        lax.fori_loop(0, TASKS_PER_W, body, 0)

        drain_out((TASKS_PER_W - 2) % 2)
        drain_out((TASKS_PER_W - 1) % 2)

    return k


def kernel(x, tables):
    cont_t = jnp.transpose(x[:, :, :OFFSET], (1, 2, 0))
    idx_t = jnp.transpose(x[:, :, OFFSET:].astype(jnp.int32), (1, 2, 0))
    idx = idx_t.reshape(-1)
    emb4 = _build_sc_kernel()(tables, idx)
    emb5 = emb4.reshape(S, D_EMB // 8, B // 128, 8, 128)
    emb_t = jnp.transpose(emb5, (0, 1, 3, 2, 4)).reshape(S, D_EMB, B)
    full = jnp.concatenate([cont_t, emb_t], axis=1)
    return jnp.transpose(full, (2, 0, 1))
